# baseline (speedup 1.0000x reference)
import functools

import jax
import jax.numpy as jnp
from jax import lax
from jax.experimental import pallas as pl
from jax.experimental.pallas import tpu as pltpu
from jax.experimental.pallas import tpu_sc as plsc

N = 32768
H = 768
L = 16
HV = H // L
NC, NS = 2, 16
NW = NC * NS
NPW = N // NW
CT = 4
G = 10
GR = G * CT
NCH = NPW // CT
NQ = NCH // 4
INV_H = 1.0 / H
EPS = 1e-12

_SIZES = (30522, 512, 2, 128, 1024, 1024, 1024, 1024)
_OFF = []
_acc = 0
for _s in _SIZES:
    _OFF.append(_acc)
    _acc += _s
R_TOTAL = _acc


def _rsqrt(x):
    xi = lax.bitcast_convert_type(x, jnp.int32)
    y = lax.bitcast_convert_type(0x5F3759DF - (xi >> 1), jnp.float32)
    for _ in range(3):
        y = y * (1.5 - 0.5 * x * y * y)
    return y


def _reduce_splat(v):
    dnums = lax.GatherDimensionNumbers(
        offset_dims=(), collapsed_slice_dims=(0,), start_index_map=(0,))
    for off in (8, 4, 2, 1):
        perm = lax.iota(jnp.int32, L) ^ off
        v = v + lax.gather(v, perm[:, None], dnums, (1,),
                           mode=lax.GatherScatterMode.PROMISE_IN_BOUNDS)
    return v


def _body(mega_hbm, idx_hbm, gb_hbm, out_hbm,
          idx_v, buf_a, buf_b, obuf, gb_v, s_a, s_b, s_o0, s_o1):
    wid = lax.axis_index("s") * NC + lax.axis_index("c")
    base = wid * NPW
    pltpu.sync_copy(gb_hbm, gb_v)
    pltpu.sync_copy(idx_hbm.at[pl.ds(wid * (NPW * G), NPW * G)], idx_v)

    def gather(c, buf, sem):
        pltpu.async_copy(mega_hbm.at[idx_v.at[pl.ds(c * GR, GR)]], buf, sem)

    def gather_wait(c, buf, sem):
        pltpu.make_async_copy(
            mega_hbm.at[idx_v.at[pl.ds(c * GR, GR)]], buf, sem).wait()

    def out_desc(p, slot, sem):
        return pltpu.make_async_copy(
            obuf.at[slot], out_hbm.at[pl.ds(base + p * 8, 8)], sem)

    def token(buf, t, slot, j):
        def p1(i, carry):
            s, q = carry
            sl = pl.ds(i * L, L)
            v = buf[t, sl]
            for k in range(1, G):
                v = v + buf[k * CT + t, sl]
            obuf[slot, j, sl] = v
            return s + v, q + v * v
        z = jnp.zeros((L,), jnp.float32)
        s, q = lax.fori_loop(0, HV, p1, (z, z))
        mu = _reduce_splat(s) * INV_H
        var = _reduce_splat(q) * INV_H - mu * mu
        rstd = _rsqrt(var + EPS)

        def p2(i, _):
            sl = pl.ds(i * L, L)
            w = obuf[slot, j, sl]
            obuf[slot, j, sl] = (w - mu) * rstd * gb_v[0, sl] + gb_v[1, sl]
            return 0
        lax.fori_loop(0, HV, p2, 0)

    def chunk(buf, c, slot, j0):
        for t in range(CT):
            token(buf, t, slot, j0 + t)

    def body(q, _):
        c0 = 4 * q

        @pl.when(q >= 1)
        def _():
            out_desc(2 * q - 2, 0, s_o0).wait()
        gather_wait(c0, buf_a, s_a)
        chunk(buf_a, c0, 0, 0)
        gather(c0 + 2, buf_a, s_a)
        gather_wait(c0 + 1, buf_b, s_b)
        chunk(buf_b, c0 + 1, 0, 4)
        gather(c0 + 3, buf_b, s_b)
        out_desc(2 * q, 0, s_o0).start()

        @pl.when(q >= 1)
        def _():
            out_desc(2 * q - 1, 1, s_o1).wait()
        gather_wait(c0 + 2, buf_a, s_a)
        chunk(buf_a, c0 + 2, 1, 0)

        @pl.when(q < NQ - 1)
        def _():
            gather(c0 + 4, buf_a, s_a)
        gather_wait(c0 + 3, buf_b, s_b)
        chunk(buf_b, c0 + 3, 1, 4)

        @pl.when(q < NQ - 1)
        def _():
            gather(c0 + 5, buf_b, s_b)
        out_desc(2 * q + 1, 1, s_o1).start()
        return 0

    gather(0, buf_a, s_a)
    gather(1, buf_b, s_b)
    lax.fori_loop(0, NQ, body, 0)
    out_desc(2 * NQ - 2, 0, s_o0).wait()
    out_desc(2 * NQ - 1, 1, s_o1).wait()


@functools.cache
def _build():
    mesh = plsc.VectorSubcoreMesh(core_axis_name="c", subcore_axis_name="s",
                                  num_cores=NC, num_subcores=NS)
    return pl.kernel(
        _body,
        out_type=jax.ShapeDtypeStruct((N, H), jnp.float32),
        mesh=mesh,
        scratch_types=[
            pltpu.VMEM((NPW * G,), jnp.int32),
            pltpu.VMEM((GR, H), jnp.float32),
            pltpu.VMEM((GR, H), jnp.float32),
            pltpu.VMEM((2, 8, H), jnp.float32),
            pltpu.VMEM((2, H), jnp.float32),
            pltpu.SemaphoreType.DMA,
            pltpu.SemaphoreType.DMA,
            pltpu.SemaphoreType.DMA,
            pltpu.SemaphoreType.DMA,
        ],
    )


def kernel(input_ids, bbox, token_type_ids, position_ids, font_ids,
           word_emb, pos_emb, x_emb, y_emb, h_emb, w_emb, tok_emb, font_emb,
           gamma, beta):
    B, S = input_ids.shape
    i32 = jnp.int32
    mega = jnp.concatenate([word_emb, pos_emb, tok_emb, font_emb,
                            x_emb, y_emb, h_emb, w_emb], axis=0)
    ids = input_ids.reshape(N).astype(i32)
    pos_idx = jnp.broadcast_to(position_ids, (B, S)).reshape(N).astype(i32)
    tok_idx = token_type_ids.reshape(N).astype(i32)
    font_idx = font_ids.reshape(N).astype(i32)
    bb = bbox.astype(i32)
    left = bb[:, :, 0].reshape(N)
    upper = bb[:, :, 1].reshape(N)
    right = bb[:, :, 2].reshape(N)
    lower = bb[:, :, 3].reshape(N)
    idx10 = jnp.stack([
        ids,
        pos_idx + _OFF[1],
        tok_idx + _OFF[2],
        font_idx + _OFF[3],
        left + _OFF[4],
        upper + _OFF[5],
        right + _OFF[4],
        lower + _OFF[5],
        (lower - upper) + _OFF[6],
        (right - left) + _OFF[7],
    ])
    idx = idx10.reshape(G, NW, NCH, CT).transpose(1, 2, 0, 3).reshape(-1)
    gb = jnp.stack([gamma, beta])
    out = _build()(mega, idx, gb)
    return out.reshape(B, S, H)
